# baseline (device time: 112798 ns/iter reference)
import jax
import jax.numpy as jnp
from jax import lax
from jax.experimental import pallas as pl
from jax.experimental.pallas import tpu as pltpu


def kernel(O, Wo):
    B, S, H_loc, D = O.shape
    K = H_loc * D
    N = Wo.shape[1]
    s_half = S // 2

    X = O.reshape(B, S, K)

    def body(x_ref, wo_ref, out_ref, send_buf, recv_buf, send_sem, recv_sem):
        my_x = lax.axis_index("x")
        my_y = lax.axis_index("y")
        peer_x = 1 - my_x

        barrier_sem = pltpu.get_barrier_semaphore()
        pl.semaphore_signal(
            barrier_sem, inc=1,
            device_id=(peer_x, my_y), device_id_type=pl.DeviceIdType.MESH,
        )
        pl.semaphore_wait(barrier_sem, 1)

        own_start = my_x * s_half
        peer_start = peer_x * s_half

        for b in range(B):
            send_buf[b, :, :] = jnp.dot(
                x_ref[b, pl.ds(peer_start, s_half), :], wo_ref[:, :],
                preferred_element_type=jnp.float32,
            )

        rdma = pltpu.make_async_remote_copy(
            src_ref=send_buf,
            dst_ref=recv_buf,
            send_sem=send_sem,
            recv_sem=recv_sem,
            device_id=(peer_x, my_y),
            device_id_type=pl.DeviceIdType.MESH,
        )
        rdma.start()

        for b in range(B):
            out_ref[b, :, :] = jnp.dot(
                x_ref[b, pl.ds(own_start, s_half), :], wo_ref[:, :],
                preferred_element_type=jnp.float32,
            )

        rdma.wait()
        for b in range(B):
            out_ref[b, :, :] = out_ref[b, :, :] + recv_buf[b, :, :]

    return pl.pallas_call(
        body,
        out_shape=jax.ShapeDtypeStruct((B, s_half, N), jnp.float32),
        in_specs=[
            pl.BlockSpec(memory_space=pltpu.VMEM),
            pl.BlockSpec(memory_space=pltpu.VMEM),
        ],
        out_specs=pl.BlockSpec(memory_space=pltpu.VMEM),
        scratch_shapes=[
            pltpu.VMEM((B, s_half, N), jnp.float32),
            pltpu.VMEM((B, s_half, N), jnp.float32),
            pltpu.SemaphoreType.DMA,
            pltpu.SemaphoreType.DMA,
        ],
        compiler_params=pltpu.CompilerParams(collective_id=0),
    )(X, Wo)


# device time: 73472 ns/iter; 1.5353x vs baseline; 1.5353x over previous
import jax
import jax.numpy as jnp
from jax import lax
from jax.experimental import pallas as pl
from jax.experimental.pallas import tpu as pltpu

N_CHUNKS = 8


def kernel(O, Wo):
    B, S, H_loc, D = O.shape
    K = H_loc * D
    N = Wo.shape[1]
    s_half = S // 2
    n_half = N // 2
    rows = s_half // N_CHUNKS

    X = O.reshape(B, S, K)

    def body(x_ref, wo_ref, out_ref, send_buf, recv_buf,
             x_send_sems, x_recv_sems, y_send_sems, y_recv_sems):
        my_x = lax.axis_index("x")
        my_y = lax.axis_index("y")
        x_peer = (1 - my_x, my_y)
        y_peer = (my_x, 1 - my_y)

        barrier_sem = pltpu.get_barrier_semaphore()
        for nbr in (x_peer, y_peer):
            pl.semaphore_signal(
                barrier_sem, inc=1,
                device_id=nbr, device_id_type=pl.DeviceIdType.MESH,
            )
        pl.semaphore_wait(barrier_sem, 2)

        own_rows = my_x * s_half
        peer_rows = (1 - my_x) * s_half
        my_col = my_y * n_half

        def x_rdma(c):
            return pltpu.make_async_remote_copy(
                src_ref=send_buf.at[:, pl.ds(c * rows, rows), :],
                dst_ref=recv_buf.at[:, pl.ds(c * rows, rows), :],
                send_sem=x_send_sems.at[c],
                recv_sem=x_recv_sems.at[c],
                device_id=x_peer,
                device_id_type=pl.DeviceIdType.MESH,
            )

        def y_rdma(c):
            return pltpu.make_async_remote_copy(
                src_ref=out_ref.at[:, pl.ds(c * rows, rows), pl.ds(my_col, n_half)],
                dst_ref=out_ref.at[:, pl.ds(c * rows, rows), pl.ds(my_col, n_half)],
                send_sem=y_send_sems.at[c],
                recv_sem=y_recv_sems.at[c],
                device_id=y_peer,
                device_id_type=pl.DeviceIdType.MESH,
            )

        for c in range(N_CHUNKS):
            r0 = c * rows
            for b in range(B):
                send_buf[b, pl.ds(r0, rows), :] = jnp.dot(
                    x_ref[b, pl.ds(peer_rows + r0, rows), :],
                    wo_ref[:, pl.ds(my_col, n_half)],
                    preferred_element_type=jnp.float32,
                )
            x_rdma(c).start()

        for b in range(B):
            out_ref[b, :, pl.ds(my_col, n_half)] = jnp.dot(
                x_ref[b, pl.ds(own_rows, s_half), :],
                wo_ref[:, pl.ds(my_col, n_half)],
                preferred_element_type=jnp.float32,
            )

        for c in range(N_CHUNKS):
            r0 = c * rows
            x_rdma(c).wait_recv()
            for b in range(B):
                out_ref[b, pl.ds(r0, rows), pl.ds(my_col, n_half)] = (
                    out_ref[b, pl.ds(r0, rows), pl.ds(my_col, n_half)]
                    + recv_buf[b, pl.ds(r0, rows), :]
                )
            y_rdma(c).start()

        for c in range(N_CHUNKS):
            y_rdma(c).wait_recv()
        for c in range(N_CHUNKS):
            x_rdma(c).wait_send()
            y_rdma(c).wait_send()

    return pl.pallas_call(
        body,
        out_shape=jax.ShapeDtypeStruct((B, s_half, N), jnp.float32),
        in_specs=[
            pl.BlockSpec(memory_space=pltpu.VMEM),
            pl.BlockSpec(memory_space=pltpu.VMEM),
        ],
        out_specs=pl.BlockSpec(memory_space=pltpu.VMEM),
        scratch_shapes=[
            pltpu.VMEM((B, s_half, n_half), jnp.float32),
            pltpu.VMEM((B, s_half, n_half), jnp.float32),
            pltpu.SemaphoreType.DMA((N_CHUNKS,)),
            pltpu.SemaphoreType.DMA((N_CHUNKS,)),
            pltpu.SemaphoreType.DMA((N_CHUNKS,)),
            pltpu.SemaphoreType.DMA((N_CHUNKS,)),
        ],
        compiler_params=pltpu.CompilerParams(collective_id=0),
    )(X, Wo)
